# baseline (speedup 1.0000x reference)
import functools

import jax
import jax.numpy as jnp
from jax import lax
from jax.experimental import pallas as pl
from jax.experimental.pallas import tpu as pltpu
from jax.experimental.pallas import tpu_sc as plsc

N_NODES = 10000
N_EDGES = 320000
D_H = 128

NC = 2
NS = 16
NW = NC * NS
CHK = 128
NCH = N_EDGES // CHK
BCW = NCH // NW
XTRA = NCH - BCW * NW
KF = 13
SUP = BCW // KF

_relu = lambda x: jnp.maximum(x, 0.0)


def _dot(a, b):
    return jnp.dot(a, b, preferred_element_type=jnp.float32)


def _sc_gather_sum(table_a, table_b, idx_a2d, idx_b2d):
    mesh = plsc.VectorSubcoreMesh(core_axis_name="c", subcore_axis_name="s")

    @functools.partial(
        pl.kernel, mesh=mesh,
        compiler_params=pltpu.CompilerParams(use_tc_tiling_on_sc=False),
        out_type=jax.ShapeDtypeStruct((N_EDGES, 16), jnp.float32),
        scratch_types=[pltpu.VMEM((BCW + 1, CHK), jnp.int32),
                       pltpu.VMEM((BCW + 1, CHK), jnp.int32),
                       pltpu.VMEM((KF * CHK, 16), jnp.float32),
                       pltpu.SemaphoreType.DMA],
    )
    def gth(ta, tb, ia, ib, out, ia_v, ib_v, rbuf, sem):
        wid = lax.axis_index("s") * NC + lax.axis_index("c")
        s_w = BCW * wid + jnp.minimum(wid, XTRA)
        pltpu.sync_copy(ia.at[pl.ds(s_w, BCW)], ia_v.at[pl.ds(0, BCW)])
        pltpu.sync_copy(ib.at[pl.ds(s_w, BCW)], ib_v.at[pl.ds(0, BCW)])

        @pl.when(wid < XTRA)
        def _():
            pltpu.sync_copy(ia.at[pl.ds(s_w + BCW, 1)], ia_v.at[pl.ds(BCW, 1)])
            pltpu.sync_copy(ib.at[pl.ds(s_w + BCW, 1)], ib_v.at[pl.ds(BCW, 1)])

        def sup(g, carry):
            cb = g * KF
            fired = [pltpu.async_copy(ta.at[ia_v.at[cb + t]],
                                      rbuf.at[pl.ds(t * CHK, CHK)], sem)
                     for t in range(KF)]
            for d in fired:
                d.wait()
            fired = [pltpu.async_copy(tb.at[ib_v.at[cb + t]],
                                      rbuf.at[pl.ds(t * CHK, CHK)], sem, add=True)
                     for t in range(KF)]
            for d in fired:
                d.wait()
            pltpu.sync_copy(rbuf, out.at[pl.ds((s_w + cb) * CHK, KF * CHK)])
            return carry

        lax.fori_loop(0, SUP, sup, 0)

        @pl.when(wid < XTRA)
        def _():
            pltpu.async_copy(ta.at[ia_v.at[BCW]],
                             rbuf.at[pl.ds(0, CHK)], sem).wait()
            pltpu.async_copy(tb.at[ib_v.at[BCW]],
                             rbuf.at[pl.ds(0, CHK)], sem, add=True).wait()
            pltpu.sync_copy(rbuf.at[pl.ds(0, CHK)],
                            out.at[pl.ds((s_w + BCW) * CHK, CHK)])

    return gth(table_a, table_b, idx_a2d, idx_b2d)


def _sc_scatter_add(rows, idx2d, zeros_stripe):
    mesh = plsc.VectorSubcoreMesh(core_axis_name="c", subcore_axis_name="s")
    stripe = N_NODES // NS

    @functools.partial(
        pl.kernel, mesh=mesh,
        compiler_params=pltpu.CompilerParams(use_tc_tiling_on_sc=False),
        out_type=jax.ShapeDtypeStruct((NC, N_NODES, 16), jnp.float32),
        scratch_types=[pltpu.VMEM((BCW + 1, CHK), jnp.int32),
                       pltpu.VMEM((KF * CHK, 16), jnp.float32),
                       pltpu.VMEM_SHARED((N_NODES, 16), jnp.float32),
                       pltpu.SemaphoreType.DMA],
    )
    def sct(rows_h, idx_h, zro_h, out_h, idx_v, rbuf, acc, sem):
        cid = lax.axis_index("c")
        sid = lax.axis_index("s")
        wid = sid * NC + cid
        s_w = BCW * wid + jnp.minimum(wid, XTRA)
        pltpu.sync_copy(zro_h, acc.at[pl.ds(sid * stripe, stripe)])
        pltpu.sync_copy(idx_h.at[pl.ds(s_w, BCW)], idx_v.at[pl.ds(0, BCW)])

        @pl.when(wid < XTRA)
        def _():
            pltpu.sync_copy(idx_h.at[pl.ds(s_w + BCW, 1)], idx_v.at[pl.ds(BCW, 1)])

        plsc.subcore_barrier()

        def sup(g, carry):
            cb = g * KF
            pltpu.sync_copy(rows_h.at[pl.ds((s_w + cb) * CHK, KF * CHK)], rbuf)
            fired = [pltpu.async_copy(rbuf.at[pl.ds(t * CHK, CHK)],
                                      acc.at[idx_v.at[cb + t]], sem, add=True)
                     for t in range(KF)]
            for d in fired:
                d.wait()
            return carry

        lax.fori_loop(0, SUP, sup, 0)

        @pl.when(wid < XTRA)
        def _():
            pltpu.sync_copy(rows_h.at[pl.ds((s_w + BCW) * CHK, CHK)],
                            rbuf.at[pl.ds(0, CHK)])
            pltpu.async_copy(rbuf.at[pl.ds(0, CHK)],
                             acc.at[idx_v.at[BCW]], sem, add=True).wait()

        plsc.subcore_barrier()
        pltpu.sync_copy(acc.at[pl.ds(sid * stripe, stripe)],
                        out_h.at[cid, pl.ds(sid * stripe, stripe)])

    return sct(rows, idx2d, zeros_stripe)


def _full(shape):
    nd = len(shape)
    return pl.BlockSpec(shape, lambda i: (0,) * nd)


def _tc_node_pre(hs, xs, w0r, w0s, wh0):
    bn, grid = 1000, N_NODES // 1000

    def body(hs_r, xs_r, wr_r, ws_r, wh_r, tr_r, ts_r, hh_r):
        h = hs_r[...]
        x = xs_r[...]
        z = jnp.zeros((bn, 5), jnp.float32)
        tr_r[...] = jnp.concatenate(
            [_dot(h, wr_r[...]), x, z], axis=1)
        ts_r[...] = jnp.concatenate(
            [_dot(h, ws_r[...]), -x, z], axis=1)
        hh_r[...] = _dot(h, wh_r[...])

    return pl.pallas_call(
        body,
        grid=(grid,),
        in_specs=[pl.BlockSpec((bn, D_H), lambda i: (i, 0)),
                  pl.BlockSpec((bn, 3), lambda i: (i, 0)),
                  _full((D_H, 8)), _full((D_H, 8)), _full((D_H, 8))],
        out_specs=[pl.BlockSpec((bn, 16), lambda i: (i, 0)),
                   pl.BlockSpec((bn, 16), lambda i: (i, 0)),
                   pl.BlockSpec((bn, 8), lambda i: (i, 0))],
        out_shape=[jax.ShapeDtypeStruct((N_NODES, 16), jnp.float32),
                   jax.ShapeDtypeStruct((N_NODES, 16), jnp.float32),
                   jax.ShapeDtypeStruct((N_NODES, 8), jnp.float32)],
    )(hs, xs, w0r, w0s, wh0)


NP8 = N_EDGES // 8


def _tc_edge0(comb_p, edges_p, m_msg, m_sum, m2k, m3k, m1m,
              b0t, w0mt, b2t, b3t, be1t, w1mt):
    be, grid = 2000, NP8 // 2000

    def body(x_ref, ed_ref, mmsg_r, msum_r, mk2_r, mk3_r, m1m_r,
             b0_r, w0m_r, b2_r, b3_r, be1_r, w1m_r, mp_r, e1_r):
        x = x_ref[...]
        p = x * x
        mm = _dot(p, msum_r[...])
        e0 = _dot(ed_ref[...], mmsg_r[...])
        pre = x + mm * w0m_r[...] + e0 + b0_r[...]
        h = _relu(pre)
        h = _relu(_dot(h, mk2_r[...]) + b2_r[...])
        msg = _dot(h, mk3_r[...]) + b3_r[...]
        m2v8 = p + jnp.roll(p, -1, axis=1) + jnp.roll(p, -2, axis=1)
        lane = jax.lax.broadcasted_iota(jnp.int32, (be, 128), 1) % 16
        mp_r[...] = msg + jnp.where(lane == 8, m2v8, 0.0)
        e1_r[...] = (_dot(msg, m1m_r[...])
                     + mm * w1m_r[...] + be1_r[...])

    return pl.pallas_call(
        body,
        grid=(grid,),
        in_specs=[pl.BlockSpec((be, 128), lambda i: (i, 0)),
                  pl.BlockSpec((be, 128), lambda i: (i, 0)),
                  _full((128, 128)), _full((128, 128)), _full((128, 128)),
                  _full((128, 128)), _full((128, 128)),
                  _full((1, 128)), _full((1, 128)), _full((1, 128)),
                  _full((1, 128)), _full((1, 128)), _full((1, 128))],
        out_specs=[pl.BlockSpec((be, 128), lambda i: (i, 0)),
                   pl.BlockSpec((be, 128), lambda i: (i, 0))],
        out_shape=[jax.ShapeDtypeStruct((NP8, 128), jnp.float32),
                   jax.ShapeDtypeStruct((NP8, 128), jnp.float32)],
    )(comb_p, edges_p, m_msg, m_sum, m2k, m3k, m1m,
      b0t, w0mt, b2t, b3t, be1t, w1mt)


def _tc_node0(aggp, hh0, wha, bh, wh2, bh2, wh3, bh3, w1er, w1es, wh1hs):
    bn, grid = 2000, N_NODES // 2000

    def body(ag_r, hh_r, wha_r, bh_r, wh2_r, bh2_r, wh3_r, bh3_r,
             w1er_r, w1es_r, wh1_r, t1r_r, t1s_r, hh1_r):
        a8 = (ag_r[0] + ag_r[1])[:, 0:8]
        pre = hh_r[...] + _dot(a8, wha_r[...]) + bh_r[...]
        h = _relu(pre)
        h = _relu(_dot(h, wh2_r[...]) + bh2_r[...])
        hs1 = _dot(h, wh3_r[...]) + bh3_r[...]
        z8 = jnp.zeros((bn, 8), jnp.float32)
        t1r_r[...] = jnp.concatenate(
            [_dot(hs1, w1er_r[...]), z8], axis=1)
        t1s_r[...] = jnp.concatenate(
            [z8, _dot(hs1, w1es_r[...])], axis=1)
        hh1_r[...] = _dot(hs1, wh1_r[...])

    return pl.pallas_call(
        body,
        grid=(grid,),
        in_specs=[pl.BlockSpec((NC, bn, 16), lambda i: (0, i, 0)),
                  pl.BlockSpec((bn, 8), lambda i: (i, 0)),
                  _full((8, 8)), _full((1, 8)), _full((8, 8)), _full((1, 8)),
                  _full((8, 8)), _full((1, 8)),
                  _full((8, 8)), _full((8, 8)), _full((8, 8))],
        out_specs=[pl.BlockSpec((bn, 16), lambda i: (i, 0)),
                   pl.BlockSpec((bn, 16), lambda i: (i, 0)),
                   pl.BlockSpec((bn, 8), lambda i: (i, 0))],
        out_shape=[jax.ShapeDtypeStruct((N_NODES, 16), jnp.float32),
                   jax.ShapeDtypeStruct((N_NODES, 16), jnp.float32),
                   jax.ShapeDtypeStruct((N_NODES, 8), jnp.float32)],
    )(aggp, hh0, wha, bh, wh2, bh2, wh3, bh3, w1er, w1es, wh1hs)


def _tc_edge1(comb1_p, e1f_p, m2k, m3k, b2t, b3t):
    be, grid = 2000, NP8 // 2000

    def body(c_ref, e1_r, m2_r, m3_r, b2_r, b3_r, mp_r):
        c = c_ref[...]
        pre = c + jnp.roll(c, -8, axis=1) + e1_r[...]
        h = _relu(pre)
        h = _relu(_dot(h, m2_r[...]) + b2_r[...])
        mp_r[...] = _dot(h, m3_r[...]) + b3_r[...]

    return pl.pallas_call(
        body,
        grid=(grid,),
        in_specs=[pl.BlockSpec((be, 128), lambda i: (i, 0)),
                  pl.BlockSpec((be, 128), lambda i: (i, 0)),
                  _full((128, 128)), _full((128, 128)),
                  _full((1, 128)), _full((1, 128))],
        out_specs=[pl.BlockSpec((be, 128), lambda i: (i, 0))],
        out_shape=[jax.ShapeDtypeStruct((NP8, 128), jnp.float32)],
    )(comb1_p, e1f_p, m2k, m3k, b2t, b3t)[0]


def _tc_node1(aggp, hh1, wha, bh, wh2, bh2, wh3, bh3):
    bn, grid = 2000, N_NODES // 2000

    def body(ag_r, hh_r, wha_r, bh_r, wh2_r, bh2_r, wh3_r, bh3_r, agg_r, hs_r):
        a8 = (ag_r[0] + ag_r[1])[:, 0:8]
        agg_r[...] = a8
        pre = hh_r[...] + _dot(a8, wha_r[...]) + bh_r[...]
        h = _relu(pre)
        h = _relu(_dot(h, wh2_r[...]) + bh2_r[...])
        hs_r[...] = _dot(h, wh3_r[...]) + bh3_r[...]

    return pl.pallas_call(
        body,
        grid=(grid,),
        in_specs=[pl.BlockSpec((NC, bn, 16), lambda i: (0, i, 0)),
                  pl.BlockSpec((bn, 8), lambda i: (i, 0)),
                  _full((8, 8)), _full((1, 8)), _full((8, 8)), _full((1, 8)),
                  _full((8, 8)), _full((1, 8))],
        out_specs=[pl.BlockSpec((bn, 8), lambda i: (i, 0)),
                   pl.BlockSpec((bn, 8), lambda i: (i, 0))],
        out_shape=[jax.ShapeDtypeStruct((N_NODES, 8), jnp.float32),
                   jax.ShapeDtypeStruct((N_NODES, 8), jnp.float32)],
    )(aggp, hh1, wha, bh, wh2, bh2, wh3, bh3)


def kernel(xs, vs, hs, edges, receivers, senders, node_mask, edge_mask,
           e_params, h_params):
    f32 = jnp.float32
    w0e, b0e = e_params[0][0]
    w0r, w0s = w0e[0:D_H], w0e[D_H:2 * D_H]
    w0m, w0msg = w0e[2 * D_H:2 * D_H + 1], w0e[2 * D_H + 1:]
    w2e0, b2e0 = e_params[0][1]
    w3e0, b3e0 = e_params[0][2]

    w1e, b1e = e_params[1][0]
    w1er, w1es = w1e[0:8], w1e[8:16]
    w1m, w1msg = w1e[16:17], w1e[17:25]
    w2e1, b2e1 = e_params[1][1]
    w3e1, b3e1 = e_params[1][2]

    wh0, bh0 = h_params[0][0]
    wh0hs, wh0a = wh0[0:D_H], wh0[D_H:]
    wh02, bh02 = h_params[0][1]
    wh03, bh03 = h_params[0][2]

    wh1, bh1 = h_params[1][0]
    wh1hs, wh1a = wh1[0:8], wh1[8:16]
    wh12, bh12 = h_params[1][1]
    wh13, bh13 = h_params[1][2]

    row = lambda v: v.reshape(1, -1).astype(f32)
    zeros_stripe = jnp.zeros((N_NODES // NS, 16), f32)
    recv2d = receivers.reshape(NCH, CHK)
    send2d = senders.reshape(NCH, CHK)

    eye8 = jnp.eye(8, dtype=f32)
    def blk(rows, cols, w):
        b = jnp.zeros((16, 16), f32).at[rows, cols].set(w)
        return jnp.kron(eye8, b)
    def tile8(v):
        return jnp.tile(jnp.concatenate([v.reshape(-1), jnp.zeros(8, f32)]), 8)[None, :]

    m_msg0 = blk(slice(0, 16), slice(0, 8), w0msg)
    m_sum = blk(slice(8, 11), slice(0, 8), 1.0)
    m2k_0 = blk(slice(0, 8), slice(0, 8), w2e0)
    m3k_0 = blk(slice(0, 8), slice(0, 8), w3e0)
    m1m_k = blk(slice(0, 8), slice(0, 8), w1msg)
    m2k_1 = blk(slice(0, 8), slice(0, 8), w2e1)
    m3k_1 = blk(slice(0, 8), slice(0, 8), w3e1)

    tr, ts, hh0 = _tc_node_pre(hs, xs, w0r, w0s, wh0hs)
    comb0 = _sc_gather_sum(tr, ts, recv2d, send2d)
    mp0_p, e1f_p = _tc_edge0(
        comb0.reshape(NP8, 128), edges.reshape(NP8, 128),
        m_msg0, m_sum, m2k_0, m3k_0, m1m_k,
        tile8(b0e), tile8(w0m), tile8(b2e0), tile8(b3e0),
        tile8(b1e), tile8(w1m))
    displacements = comb0[:, 8:11]
    mp0_r16 = mp0_p.reshape(N_EDGES, 16)
    m2 = mp0_r16[:, 8:9]
    agg0p = _sc_scatter_add(mp0_r16, recv2d, zeros_stripe)
    t1r, t1s, hh1 = _tc_node0(agg0p, hh0, wh0a, row(bh0), wh02, row(bh02),
                              wh03, row(bh03), w1er, w1es, wh1hs)
    comb1 = _sc_gather_sum(t1r, t1s, recv2d, send2d)
    mp1_p = _tc_edge1(comb1.reshape(NP8, 128), e1f_p,
                      m2k_1, m3k_1, tile8(b2e1), tile8(b3e1))
    mp1_r16 = mp1_p.reshape(N_EDGES, 16)
    messages = mp1_r16[:, 0:8]
    agg1p = _sc_scatter_add(mp1_r16, recv2d, zeros_stripe)
    agg1, hs_out = _tc_node1(agg1p, hh1, wh1a, row(bh1), wh12, row(bh12),
                             wh13, row(bh13))

    return (hs_out, messages, agg1, displacements, m2)


# speedup vs baseline: 1.2428x; 1.2428x over previous
import functools

import jax
import jax.numpy as jnp
from jax import lax
from jax.experimental import pallas as pl
from jax.experimental.pallas import tpu as pltpu
from jax.experimental.pallas import tpu_sc as plsc

N_NODES = 10000
N_EDGES = 320000
D_H = 128

NC = 2
NS = 16
NW = NC * NS
CHK = 128
NCH = N_EDGES // CHK
BCW = NCH // NW
XTRA = NCH - BCW * NW
KF = 13
SUP = BCW // KF

_relu = lambda x: jnp.maximum(x, 0.0)


def _dot(a, b):
    return jnp.dot(a, b, preferred_element_type=jnp.float32)


def _sc_gather_sum(table_a, table_b, idx_a2d, idx_b2d):
    mesh = plsc.VectorSubcoreMesh(core_axis_name="c", subcore_axis_name="s")

    @functools.partial(
        pl.kernel, mesh=mesh,
        compiler_params=pltpu.CompilerParams(use_tc_tiling_on_sc=False),
        out_type=jax.ShapeDtypeStruct((N_EDGES, 16), jnp.float32),
        scratch_types=[pltpu.VMEM((BCW + 1, CHK), jnp.int32),
                       pltpu.VMEM((BCW + 1, CHK), jnp.int32),
                       pltpu.VMEM((KF * CHK, 16), jnp.float32),
                       pltpu.SemaphoreType.DMA],
    )
    def gth(ta, tb, ia, ib, out, ia_v, ib_v, rbuf, sem):
        wid = lax.axis_index("s") * NC + lax.axis_index("c")
        s_w = BCW * wid + jnp.minimum(wid, XTRA)
        pltpu.sync_copy(ia.at[pl.ds(s_w, BCW)], ia_v.at[pl.ds(0, BCW)])
        pltpu.sync_copy(ib.at[pl.ds(s_w, BCW)], ib_v.at[pl.ds(0, BCW)])

        @pl.when(wid < XTRA)
        def _():
            pltpu.sync_copy(ia.at[pl.ds(s_w + BCW, 1)], ia_v.at[pl.ds(BCW, 1)])
            pltpu.sync_copy(ib.at[pl.ds(s_w + BCW, 1)], ib_v.at[pl.ds(BCW, 1)])

        def sup(g, carry):
            cb = g * KF
            fired = [pltpu.async_copy(ta.at[ia_v.at[cb + t]],
                                      rbuf.at[pl.ds(t * CHK, CHK)], sem)
                     for t in range(KF)]
            for d in fired:
                d.wait()
            fired = [pltpu.async_copy(tb.at[ib_v.at[cb + t]],
                                      rbuf.at[pl.ds(t * CHK, CHK)], sem, add=True)
                     for t in range(KF)]
            for d in fired:
                d.wait()
            pltpu.sync_copy(rbuf, out.at[pl.ds((s_w + cb) * CHK, KF * CHK)])
            return carry

        lax.fori_loop(0, SUP, sup, 0)

        @pl.when(wid < XTRA)
        def _():
            pltpu.async_copy(ta.at[ia_v.at[BCW]],
                             rbuf.at[pl.ds(0, CHK)], sem).wait()
            pltpu.async_copy(tb.at[ib_v.at[BCW]],
                             rbuf.at[pl.ds(0, CHK)], sem, add=True).wait()
            pltpu.sync_copy(rbuf.at[pl.ds(0, CHK)],
                            out.at[pl.ds((s_w + BCW) * CHK, CHK)])

    return gth(table_a, table_b, idx_a2d, idx_b2d)


def _sc_scatter_add(rows, idx2d, zeros_stripe):
    mesh = plsc.VectorSubcoreMesh(core_axis_name="c", subcore_axis_name="s")
    stripe = N_NODES // NS

    @functools.partial(
        pl.kernel, mesh=mesh,
        compiler_params=pltpu.CompilerParams(use_tc_tiling_on_sc=False),
        out_type=jax.ShapeDtypeStruct((NC, N_NODES, 16), jnp.float32),
        scratch_types=[pltpu.VMEM((BCW + 1, CHK), jnp.int32),
                       pltpu.VMEM((KF * CHK, 16), jnp.float32),
                       pltpu.VMEM_SHARED((N_NODES, 16), jnp.float32),
                       pltpu.SemaphoreType.DMA],
    )
    def sct(rows_h, idx_h, zro_h, out_h, idx_v, rbuf, acc, sem):
        cid = lax.axis_index("c")
        sid = lax.axis_index("s")
        wid = sid * NC + cid
        s_w = BCW * wid + jnp.minimum(wid, XTRA)
        pltpu.sync_copy(zro_h, acc.at[pl.ds(sid * stripe, stripe)])
        pltpu.sync_copy(idx_h.at[pl.ds(s_w, BCW)], idx_v.at[pl.ds(0, BCW)])

        @pl.when(wid < XTRA)
        def _():
            pltpu.sync_copy(idx_h.at[pl.ds(s_w + BCW, 1)], idx_v.at[pl.ds(BCW, 1)])

        plsc.subcore_barrier()

        def sup(g, carry):
            cb = g * KF
            pltpu.sync_copy(rows_h.at[pl.ds((s_w + cb) * CHK, KF * CHK)], rbuf)
            fired = [pltpu.async_copy(rbuf.at[pl.ds(t * CHK, CHK)],
                                      acc.at[idx_v.at[cb + t]], sem, add=True)
                     for t in range(KF)]
            for d in fired:
                d.wait()
            return carry

        lax.fori_loop(0, SUP, sup, 0)

        @pl.when(wid < XTRA)
        def _():
            pltpu.sync_copy(rows_h.at[pl.ds((s_w + BCW) * CHK, CHK)],
                            rbuf.at[pl.ds(0, CHK)])
            pltpu.async_copy(rbuf.at[pl.ds(0, CHK)],
                             acc.at[idx_v.at[BCW]], sem, add=True).wait()

        plsc.subcore_barrier()
        pltpu.sync_copy(acc.at[pl.ds(sid * stripe, stripe)],
                        out_h.at[cid, pl.ds(sid * stripe, stripe)])

    return sct(rows, idx2d, zeros_stripe)


def _full(shape):
    nd = len(shape)
    return pl.BlockSpec(shape, lambda i: (0,) * nd)


def _tc_node_pre(hs, xs, w0r, w0s, wh0):
    bn, grid = 1000, N_NODES // 1000

    def body(hs_r, xs_r, wr_r, ws_r, wh_r, tr_r, ts_r, hh_r):
        h = hs_r[...]
        x = xs_r[...]
        z = jnp.zeros((bn, 5), jnp.float32)
        tr_r[...] = jnp.concatenate(
            [_dot(h, wr_r[...]), x, z], axis=1)
        ts_r[...] = jnp.concatenate(
            [_dot(h, ws_r[...]), -x, z], axis=1)
        hh_r[...] = _dot(h, wh_r[...])

    return pl.pallas_call(
        body,
        grid=(grid,),
        in_specs=[pl.BlockSpec((bn, D_H), lambda i: (i, 0)),
                  pl.BlockSpec((bn, 3), lambda i: (i, 0)),
                  _full((D_H, 8)), _full((D_H, 8)), _full((D_H, 8))],
        out_specs=[pl.BlockSpec((bn, 16), lambda i: (i, 0)),
                   pl.BlockSpec((bn, 16), lambda i: (i, 0)),
                   pl.BlockSpec((bn, 8), lambda i: (i, 0))],
        out_shape=[jax.ShapeDtypeStruct((N_NODES, 16), jnp.float32),
                   jax.ShapeDtypeStruct((N_NODES, 16), jnp.float32),
                   jax.ShapeDtypeStruct((N_NODES, 8), jnp.float32)],
    )(hs, xs, w0r, w0s, wh0)


NP8 = N_EDGES // 8


def _tc_edge0(comb_p, edges_p, m_msg, m_sum, m2k, m3k, m1m,
              b0t, w0mt, b2t, b3t, be1t, w1mt):
    be, grid = 2000, NP8 // 2000

    def body(x_ref, ed_ref, mmsg_r, msum_r, mk2_r, mk3_r, m1m_r,
             b0_r, w0m_r, b2_r, b3_r, be1_r, w1m_r, dm_r, m2_r, mp_r, e1_r):
        x = x_ref[...]
        p = x * x
        mm = _dot(p, msum_r[...])
        e0 = _dot(ed_ref[...], mmsg_r[...])
        pre = x + mm * w0m_r[...] + e0 + b0_r[...]
        h = _relu(pre)
        h = _relu(_dot(h, mk2_r[...]) + b2_r[...])
        msg = _dot(h, mk3_r[...]) + b3_r[...]
        mp_r[...] = msg
        e1_r[...] = (_dot(msg, m1m_r[...])
                     + mm * w1m_r[...] + be1_r[...])
        xg = x.reshape(be, 8, 16)
        dm_r[...] = xg[:, :, 8:11].reshape(be * 8, 3)
        pg = p.reshape(be, 8, 16)
        m2_r[...] = jnp.sum(pg[:, :, 8:11], axis=2).reshape(be * 8, 1)

    return pl.pallas_call(
        body,
        grid=(grid,),
        in_specs=[pl.BlockSpec((be, 128), lambda i: (i, 0)),
                  pl.BlockSpec((be, 128), lambda i: (i, 0)),
                  _full((128, 128)), _full((128, 128)), _full((128, 128)),
                  _full((128, 128)), _full((128, 128)),
                  _full((1, 128)), _full((1, 128)), _full((1, 128)),
                  _full((1, 128)), _full((1, 128)), _full((1, 128))],
        out_specs=[pl.BlockSpec((be * 8, 3), lambda i: (i, 0)),
                   pl.BlockSpec((be * 8, 1), lambda i: (i, 0)),
                   pl.BlockSpec((be, 128), lambda i: (i, 0)),
                   pl.BlockSpec((be, 128), lambda i: (i, 0))],
        out_shape=[jax.ShapeDtypeStruct((N_EDGES, 3), jnp.float32),
                   jax.ShapeDtypeStruct((N_EDGES, 1), jnp.float32),
                   jax.ShapeDtypeStruct((NP8, 128), jnp.float32),
                   jax.ShapeDtypeStruct((NP8, 128), jnp.float32)],
    )(comb_p, edges_p, m_msg, m_sum, m2k, m3k, m1m,
      b0t, w0mt, b2t, b3t, be1t, w1mt)


def _tc_node0(aggp, hh0, wha, bh, wh2, bh2, wh3, bh3, w1er, w1es, wh1hs):
    bn, grid = 2000, N_NODES // 2000

    def body(ag_r, hh_r, wha_r, bh_r, wh2_r, bh2_r, wh3_r, bh3_r,
             w1er_r, w1es_r, wh1_r, t1r_r, t1s_r, hh1_r):
        a8 = (ag_r[0] + ag_r[1])[:, 0:8]
        pre = hh_r[...] + _dot(a8, wha_r[...]) + bh_r[...]
        h = _relu(pre)
        h = _relu(_dot(h, wh2_r[...]) + bh2_r[...])
        hs1 = _dot(h, wh3_r[...]) + bh3_r[...]
        z8 = jnp.zeros((bn, 8), jnp.float32)
        t1r_r[...] = jnp.concatenate(
            [_dot(hs1, w1er_r[...]), z8], axis=1)
        t1s_r[...] = jnp.concatenate(
            [z8, _dot(hs1, w1es_r[...])], axis=1)
        hh1_r[...] = _dot(hs1, wh1_r[...])

    return pl.pallas_call(
        body,
        grid=(grid,),
        in_specs=[pl.BlockSpec((NC, bn, 16), lambda i: (0, i, 0)),
                  pl.BlockSpec((bn, 8), lambda i: (i, 0)),
                  _full((8, 8)), _full((1, 8)), _full((8, 8)), _full((1, 8)),
                  _full((8, 8)), _full((1, 8)),
                  _full((8, 8)), _full((8, 8)), _full((8, 8))],
        out_specs=[pl.BlockSpec((bn, 16), lambda i: (i, 0)),
                   pl.BlockSpec((bn, 16), lambda i: (i, 0)),
                   pl.BlockSpec((bn, 8), lambda i: (i, 0))],
        out_shape=[jax.ShapeDtypeStruct((N_NODES, 16), jnp.float32),
                   jax.ShapeDtypeStruct((N_NODES, 16), jnp.float32),
                   jax.ShapeDtypeStruct((N_NODES, 8), jnp.float32)],
    )(aggp, hh0, wha, bh, wh2, bh2, wh3, bh3, w1er, w1es, wh1hs)


def _tc_edge1(comb1_p, e1f_p, m2k, m3k, b2t, b3t):
    be, grid = 2000, NP8 // 2000

    def body(c_ref, e1_r, m2_r, m3_r, b2_r, b3_r, mp_r, msg_r):
        c = c_ref[...]
        pre = c + jnp.roll(c, -8, axis=1) + e1_r[...]
        h = _relu(pre)
        h = _relu(_dot(h, m2_r[...]) + b2_r[...])
        msg = _dot(h, m3_r[...]) + b3_r[...]
        mp_r[...] = msg
        msg_r[...] = msg.reshape(be, 8, 16)[:, :, 0:8].reshape(be * 8, 8)

    return pl.pallas_call(
        body,
        grid=(grid,),
        in_specs=[pl.BlockSpec((be, 128), lambda i: (i, 0)),
                  pl.BlockSpec((be, 128), lambda i: (i, 0)),
                  _full((128, 128)), _full((128, 128)),
                  _full((1, 128)), _full((1, 128))],
        out_specs=[pl.BlockSpec((be, 128), lambda i: (i, 0)),
                   pl.BlockSpec((be * 8, 8), lambda i: (i, 0))],
        out_shape=[jax.ShapeDtypeStruct((NP8, 128), jnp.float32),
                   jax.ShapeDtypeStruct((N_EDGES, 8), jnp.float32)],
    )(comb1_p, e1f_p, m2k, m3k, b2t, b3t)


def _tc_node1(aggp, hh1, wha, bh, wh2, bh2, wh3, bh3):
    bn, grid = 2000, N_NODES // 2000

    def body(ag_r, hh_r, wha_r, bh_r, wh2_r, bh2_r, wh3_r, bh3_r, agg_r, hs_r):
        a8 = (ag_r[0] + ag_r[1])[:, 0:8]
        agg_r[...] = a8
        pre = hh_r[...] + _dot(a8, wha_r[...]) + bh_r[...]
        h = _relu(pre)
        h = _relu(_dot(h, wh2_r[...]) + bh2_r[...])
        hs_r[...] = _dot(h, wh3_r[...]) + bh3_r[...]

    return pl.pallas_call(
        body,
        grid=(grid,),
        in_specs=[pl.BlockSpec((NC, bn, 16), lambda i: (0, i, 0)),
                  pl.BlockSpec((bn, 8), lambda i: (i, 0)),
                  _full((8, 8)), _full((1, 8)), _full((8, 8)), _full((1, 8)),
                  _full((8, 8)), _full((1, 8))],
        out_specs=[pl.BlockSpec((bn, 8), lambda i: (i, 0)),
                   pl.BlockSpec((bn, 8), lambda i: (i, 0))],
        out_shape=[jax.ShapeDtypeStruct((N_NODES, 8), jnp.float32),
                   jax.ShapeDtypeStruct((N_NODES, 8), jnp.float32)],
    )(aggp, hh1, wha, bh, wh2, bh2, wh3, bh3)


def kernel(xs, vs, hs, edges, receivers, senders, node_mask, edge_mask,
           e_params, h_params):
    f32 = jnp.float32
    w0e, b0e = e_params[0][0]
    w0r, w0s = w0e[0:D_H], w0e[D_H:2 * D_H]
    w0m, w0msg = w0e[2 * D_H:2 * D_H + 1], w0e[2 * D_H + 1:]
    w2e0, b2e0 = e_params[0][1]
    w3e0, b3e0 = e_params[0][2]

    w1e, b1e = e_params[1][0]
    w1er, w1es = w1e[0:8], w1e[8:16]
    w1m, w1msg = w1e[16:17], w1e[17:25]
    w2e1, b2e1 = e_params[1][1]
    w3e1, b3e1 = e_params[1][2]

    wh0, bh0 = h_params[0][0]
    wh0hs, wh0a = wh0[0:D_H], wh0[D_H:]
    wh02, bh02 = h_params[0][1]
    wh03, bh03 = h_params[0][2]

    wh1, bh1 = h_params[1][0]
    wh1hs, wh1a = wh1[0:8], wh1[8:16]
    wh12, bh12 = h_params[1][1]
    wh13, bh13 = h_params[1][2]

    row = lambda v: v.reshape(1, -1).astype(f32)
    zeros_stripe = jnp.zeros((N_NODES // NS, 16), f32)
    recv2d = receivers.reshape(NCH, CHK)
    send2d = senders.reshape(NCH, CHK)

    eye8 = jnp.eye(8, dtype=f32)
    def blk(rows, cols, w):
        b = jnp.zeros((16, 16), f32).at[rows, cols].set(w)
        return jnp.kron(eye8, b)
    def tile8(v):
        return jnp.tile(jnp.concatenate([v.reshape(-1), jnp.zeros(8, f32)]), 8)[None, :]

    m_msg0 = blk(slice(0, 16), slice(0, 8), w0msg)
    m_sum = blk(slice(8, 11), slice(0, 8), 1.0)
    m2k_0 = blk(slice(0, 8), slice(0, 8), w2e0)
    m3k_0 = blk(slice(0, 8), slice(0, 8), w3e0)
    m1m_k = blk(slice(0, 8), slice(0, 8), w1msg)
    m2k_1 = blk(slice(0, 8), slice(0, 8), w2e1)
    m3k_1 = blk(slice(0, 8), slice(0, 8), w3e1)

    tr, ts, hh0 = _tc_node_pre(hs, xs, w0r, w0s, wh0hs)
    comb0 = _sc_gather_sum(tr, ts, recv2d, send2d)
    displacements, m2, mp0_p, e1f_p = _tc_edge0(
        comb0.reshape(NP8, 128), edges.reshape(NP8, 128),
        m_msg0, m_sum, m2k_0, m3k_0, m1m_k,
        tile8(b0e), tile8(w0m), tile8(b2e0), tile8(b3e0),
        tile8(b1e), tile8(w1m))
    agg0p = _sc_scatter_add(mp0_p.reshape(N_EDGES, 16), recv2d, zeros_stripe)
    t1r, t1s, hh1 = _tc_node0(agg0p, hh0, wh0a, row(bh0), wh02, row(bh02),
                              wh03, row(bh03), w1er, w1es, wh1hs)
    comb1 = _sc_gather_sum(t1r, t1s, recv2d, send2d)
    mp1_p, messages = _tc_edge1(comb1.reshape(NP8, 128), e1f_p,
                                m2k_1, m3k_1, tile8(b2e1), tile8(b3e1))
    agg1p = _sc_scatter_add(mp1_p.reshape(N_EDGES, 16), recv2d, zeros_stripe)
    agg1, hs_out = _tc_node1(agg1p, hh1, wh1a, row(bh1), wh12, row(bh12),
                             wh13, row(bh13))

    return (hs_out, messages, agg1, displacements, m2)
